# baseline (speedup 1.0000x reference)
import functools

import numpy as np
import jax
import jax.numpy as jnp
from jax import lax
from jax.experimental import pallas as pl
from jax.experimental.pallas import tpu as pltpu
from jax.experimental.pallas import tpu_sc as plsc

B, L, K, NA = 4, 1024, 30, 22
KP = 32
NC, NS, LANES = 2, 16, 16
NW = NC * NS
WPB = NW // B

LT = 512
LS = L - LT

BL = 128
GI = LT // BL
Q = 4
BLQ = BL // Q

RPW = (B * LS) // NW
CH = 2
NCH = RPW // CH
RPE = (B * LT) // NW

BL2 = 128
GI2 = LS // BL2

_SEG = (np.arange(NA * NA)[:, None] // NA == np.arange(NA)[None, :]).astype(
    np.float32)

_MESH = plsc.VectorSubcoreMesh(core_axis_name="c", subcore_axis_name="s",
                               num_cores=NC, num_subcores=NS)


def _worker_id():
    return lax.axis_index("s") * NC + lax.axis_index("c")


def _sc_eaa(eidx_flat, seqs_flat):
    @functools.partial(
        pl.kernel,
        out_type=jax.ShapeDtypeStruct((B * LT * KP,), jnp.int32),
        mesh=_MESH,
        scratch_types=[
            pltpu.VMEM((RPE * KP,), jnp.int32),
            pltpu.VMEM((RPE * KP,), jnp.int32),
            pltpu.VMEM((L,), jnp.int32),
        ],
        compiler_params=pltpu.CompilerParams(needs_layout_passes=False),
    )
    def sck(eidx_h, seqs_h, out_h, eidx_v, c_v, seqs_v):
        wid = _worker_id()
        b = wid // WPB
        l0 = (wid % WPB) * RPE

        pltpu.sync_copy(seqs_h.at[pl.ds(b * L, L)], seqs_v)
        pltpu.sync_copy(eidx_h.at[pl.ds((b * L + l0) * KP, RPE * KP)], eidx_v)

        def cstage(t, carry):
            ev = eidx_v[pl.ds(t * LANES, LANES)]
            c_v[pl.ds(t * LANES, LANES)] = plsc.load_gather(seqs_v, [ev])
            return carry
        lax.fori_loop(0, (RPE * KP) // LANES, cstage, 0)

        pltpu.sync_copy(c_v, out_h.at[pl.ds(wid * RPE * KP, RPE * KP)])

    return sck(eidx_flat, seqs_flat)


def _sc_logits(etab, eidx_flat, seqs_flat):
    @functools.partial(
        pl.kernel,
        out_type=jax.ShapeDtypeStruct((B * LS * KP,), jnp.float32),
        mesh=_MESH,
        scratch_types=[
            pltpu.VMEM((CH, K, NA * NA), jnp.float32),
            pltpu.VMEM((CH, K, NA * NA), jnp.float32),
            pltpu.VMEM((RPW * KP,), jnp.int32),
            pltpu.VMEM((RPW * KP,), jnp.int32),
            pltpu.VMEM((L,), jnp.int32),
            pltpu.VMEM((RPW * KP,), jnp.float32),
            pltpu.SemaphoreType.DMA,
            pltpu.SemaphoreType.DMA,
        ],
        compiler_params=pltpu.CompilerParams(needs_layout_passes=False),
    )
    def sc_kernel(etab_h, eidx_h, seqs_h, out_h,
                  ebuf0, ebuf1, eidx_v, c_v, seqs_v, out_v, sem0, sem1):
        wid = _worker_id()
        b = wid // WPB
        l0 = LT + (wid % WPB) * RPW

        pltpu.sync_copy(seqs_h.at[pl.ds(b * L, L)], seqs_v)
        pltpu.sync_copy(eidx_h.at[pl.ds((b * L + l0) * KP, RPW * KP)], eidx_v)

        def cstage(t, carry):
            ev = eidx_v[pl.ds(t * LANES, LANES)]
            c_v[pl.ds(t * LANES, LANES)] = plsc.load_gather(seqs_v, [ev])
            return carry
        lax.fori_loop(0, (RPW * KP) // LANES, cstage, 0)

        def dma_start(g, buf, sem):
            src = etab_h.at[b, pl.ds(l0 + g * CH, CH)]
            pltpu.make_async_copy(src, buf, sem).start()

        def dma_wait(buf, sem):
            src = etab_h.at[0, pl.ds(0, CH)]
            pltpu.make_async_copy(src, buf, sem).wait()

        iota = lax.iota(jnp.int32, LANES)
        a_lo = iota * NA
        a_hi = (iota + LANES) * NA

        def compute_chunk(g, buf):
            for lr in range(CH):
                r = g * CH + lr
                acc0 = jnp.zeros((LANES,), jnp.float32)
                acc1 = jnp.zeros((LANES,), jnp.float32)
                c_lo = c_v[pl.ds(r * KP, LANES)]
                c_hi = c_v[pl.ds(r * KP + LANES, LANES)]
                i_ch = jnp.full((LANES,), lr, jnp.int32)
                for j in range(K):
                    c = c_lo[j] if j < LANES else c_hi[j - LANES]
                    i_j = jnp.full((LANES,), j, jnp.int32)
                    id0 = a_lo + c
                    id1 = jnp.minimum(a_hi + c, NA * NA - 1)
                    acc0 = acc0 + plsc.load_gather(buf, [i_ch, i_j, id0])
                    acc1 = acc1 + plsc.load_gather(buf, [i_ch, i_j, id1])
                out_v[pl.ds(r * KP, LANES)] = acc0
                out_v[pl.ds(r * KP + LANES, LANES)] = acc1

        dma_start(0, ebuf0, sem0)
        dma_start(1, ebuf1, sem1)

        def iter_body(i, carry):
            dma_wait(ebuf0, sem0)
            compute_chunk(2 * i, ebuf0)

            @pl.when(i < NCH // 2 - 1)
            def _():
                dma_start(2 * i + 2, ebuf0, sem0)

            dma_wait(ebuf1, sem1)
            compute_chunk(2 * i + 1, ebuf1)

            @pl.when(i < NCH // 2 - 1)
            def _():
                dma_start(2 * i + 3, ebuf1, sem1)

            return carry
        lax.fori_loop(0, NCH // 2, iter_body, 0)

        pltpu.sync_copy(out_v, out_h.at[pl.ds(wid * RPW * KP, RPW * KP)])

    return sc_kernel(etab, eidx_flat, seqs_flat)


def _tc_main(etab, eaa2d, seqs2d, mask2d, seg):
    def body(*refs):
        et_refs = refs[:Q]
        eaa_ref, seqs_ref, mask_ref, seg_ref, s_ref, n_ref = refs[Q:]
        bb = pl.program_id(0)
        i = pl.program_id(1)
        col = lax.broadcasted_iota(jnp.int32, (1, 1, NA * NA), 2) % NA
        blk_s = jnp.float32(0.0)
        blk_n = jnp.float32(0.0)
        for q in range(Q):
            et3 = et_refs[q][0]
            rows = pl.ds(q * BLQ, BLQ)
            c3 = eaa_ref[rows, :K][:, :, None]
            masked = jnp.where(col == c3, et3, 0.0)
            s484 = jnp.sum(masked, axis=1)
            aa = jnp.dot(s484, seg_ref[...],
                         preferred_element_type=jnp.float32)
            m = jnp.max(aa, axis=1, keepdims=True)
            lse = m + jnp.log(jnp.sum(jnp.exp(aa - m), axis=1, keepdims=True))
            lane = lax.broadcasted_iota(jnp.int32, (BLQ, NA), 1)
            pick = jnp.sum(jnp.where(lane == seqs_ref[rows], aa, 0.0),
                           axis=1, keepdims=True)
            maskc = mask_ref[rows]
            blk_s = blk_s + jnp.sum((pick - lse) * maskc)
            blk_n = blk_n + jnp.sum(maskc)

        @pl.when(i == 0)
        def _():
            s_ref[bb, 0] = 0.0
            n_ref[bb, 0] = 0.0

        s_ref[bb, 0] += blk_s
        n_ref[bb, 0] += blk_n

    etab_specs = [
        pl.BlockSpec((1, BLQ, K, NA * NA),
                     functools.partial(lambda q, b, i: (b, i * Q + q, 0, 0), q))
        for q in range(Q)
    ]
    out = pl.pallas_call(
        body,
        grid=(B, GI),
        in_specs=etab_specs + [
            pl.BlockSpec((BL, KP), lambda b, i: (b * GI + i, 0)),
            pl.BlockSpec((BL, 1), lambda b, i: (b * GI + i, 0)),
            pl.BlockSpec((BL, 1), lambda b, i: (b * GI + i, 0)),
            pl.BlockSpec((NA * NA, NA), lambda b, i: (0, 0)),
        ],
        out_specs=[
            pl.BlockSpec((B, 1), lambda b, i: (0, 0),
                         memory_space=pltpu.SMEM),
            pl.BlockSpec((B, 1), lambda b, i: (0, 0),
                         memory_space=pltpu.SMEM),
        ],
        out_shape=[
            jax.ShapeDtypeStruct((B, 1), jnp.float32),
            jax.ShapeDtypeStruct((B, 1), jnp.float32),
        ],
    )(*([etab] * Q), eaa2d, seqs2d, mask2d, seg)
    return out


def _tc_loss(aa2d, seqs2d, mask2d):
    def body(aa_ref, seqs_ref, mask_ref, s_ref, n_ref):
        bb = pl.program_id(0)
        i = pl.program_id(1)
        x = aa_ref[...]
        lane = lax.broadcasted_iota(jnp.int32, (BL2, KP), 1)
        valid = lane < NA
        xm = jnp.where(valid, x, -1e30)
        m = jnp.max(xm, axis=1, keepdims=True)
        lse = m + jnp.log(jnp.sum(jnp.exp(xm - m), axis=1, keepdims=True))
        pick = jnp.sum(jnp.where(lane == seqs_ref[...], x, 0.0),
                       axis=1, keepdims=True)
        maskc = mask_ref[...]
        blk_s = jnp.sum((pick - lse) * maskc)
        blk_n = jnp.sum(maskc)

        @pl.when(i == 0)
        def _():
            s_ref[bb, 0] = 0.0
            n_ref[bb, 0] = 0.0

        s_ref[bb, 0] += blk_s
        n_ref[bb, 0] += blk_n

    out = pl.pallas_call(
        body,
        grid=(B, GI2),
        in_specs=[
            pl.BlockSpec((BL2, KP), lambda b, i: (b * GI2 + i, 0)),
            pl.BlockSpec((BL2, 1), lambda b, i: (b * GI2 + i, 0)),
            pl.BlockSpec((BL2, 1), lambda b, i: (b * GI2 + i, 0)),
        ],
        out_specs=[
            pl.BlockSpec((B, 1), lambda b, i: (0, 0),
                         memory_space=pltpu.SMEM),
            pl.BlockSpec((B, 1), lambda b, i: (0, 0),
                         memory_space=pltpu.SMEM),
        ],
        out_shape=[
            jax.ShapeDtypeStruct((B, 1), jnp.float32),
            jax.ShapeDtypeStruct((B, 1), jnp.float32),
        ],
    )(aa2d, seqs2d, mask2d)
    return out


def kernel(etab, E_idx, seqs, x_mask):
    eidx_flat = jnp.pad(E_idx, ((0, 0), (0, 0), (0, KP - K))).reshape(-1)
    seqs_flat = seqs.reshape(-1)
    maskf = x_mask.astype(jnp.float32)

    eaa = _sc_eaa(eidx_flat, seqs_flat)

    aa_sc = _sc_logits(etab, eidx_flat, seqs_flat)
    s2, n2 = _tc_loss(
        aa_sc.reshape(B * LS, KP),
        seqs[:, LT:].reshape(B * LS, 1),
        maskf[:, LT:].reshape(B * LS, 1),
    )

    s1, n1 = _tc_main(
        etab,
        eaa.reshape(B * LT, KP),
        seqs[:, :LT].reshape(B * LT, 1),
        maskf[:, :LT].reshape(B * LT, 1),
        jnp.asarray(_SEG),
    )

    s = s1[:, 0] + s2[:, 0]
    n = n1[:, 0] + n2[:, 0]
    return -jnp.mean(s / n)


# speedup vs baseline: 1.1188x; 1.1188x over previous
import functools

import numpy as np
import jax
import jax.numpy as jnp
from jax import lax
from jax.experimental import pallas as pl
from jax.experimental.pallas import tpu as pltpu
from jax.experimental.pallas import tpu_sc as plsc

B, L, K, NA = 4, 1024, 30, 22
KP = 32
NC, NS, LANES = 2, 16, 16
NW = NC * NS
RPW = (B * L) // NW
BL = 256
GI = L // BL
Q = 1
BLQ = BL // Q

_SEG = (np.arange(NA * NA)[:, None] // NA == np.arange(NA)[None, :]).astype(
    np.float32)


def _sc_eaa(eidx_flat, seqs_flat):
    mesh = plsc.VectorSubcoreMesh(core_axis_name="c", subcore_axis_name="s",
                                  num_cores=NC, num_subcores=NS)

    @functools.partial(
        pl.kernel,
        out_type=jax.ShapeDtypeStruct((B * L * KP,), jnp.int32),
        mesh=mesh,
        scratch_types=[
            pltpu.VMEM((RPW * KP,), jnp.int32),
            pltpu.VMEM((RPW * KP,), jnp.int32),
            pltpu.VMEM((L,), jnp.int32),
        ],
        compiler_params=pltpu.CompilerParams(needs_layout_passes=False),
    )
    def sck(eidx_h, seqs_h, out_h, eidx_v, c_v, seqs_v):
        cid = lax.axis_index("c")
        sid = lax.axis_index("s")
        wid = sid * NC + cid
        b = wid // (NW // B)
        row0 = wid * RPW

        pltpu.sync_copy(seqs_h.at[pl.ds(b * L, L)], seqs_v)
        pltpu.sync_copy(eidx_h.at[pl.ds(row0 * KP, RPW * KP)], eidx_v)

        def cstage(t, carry):
            ev = eidx_v[pl.ds(t * LANES, LANES)]
            c_v[pl.ds(t * LANES, LANES)] = plsc.load_gather(seqs_v, [ev])
            return carry
        lax.fori_loop(0, (RPW * KP) // LANES, cstage, 0)

        pltpu.sync_copy(c_v, out_h.at[pl.ds(row0 * KP, RPW * KP)])

    return sck(eidx_flat, seqs_flat)


def _tc_main(etab, eaa2d, seqs2d, mask2d, seg):
    def body(*refs):
        et_refs = refs[:Q]
        eaa_ref, seqs_ref, mask_ref, seg_ref, s_ref, n_ref = refs[Q:]
        bb = pl.program_id(0)
        i = pl.program_id(1)
        col = lax.broadcasted_iota(jnp.int32, (1, 1, NA * NA), 2) % NA
        blk_s = jnp.float32(0.0)
        blk_n = jnp.float32(0.0)
        for q in range(Q):
            et3 = et_refs[q][0]
            rows = pl.ds(q * BLQ, BLQ)
            c3 = eaa_ref[rows, :K][:, :, None]
            masked = jnp.where(col == c3, et3, 0.0)
            s484 = jnp.sum(masked, axis=1)
            aa = jnp.dot(s484, seg_ref[...],
                         preferred_element_type=jnp.float32)
            m = jnp.max(aa, axis=1, keepdims=True)
            lse = m + jnp.log(jnp.sum(jnp.exp(aa - m), axis=1, keepdims=True))
            lane = lax.broadcasted_iota(jnp.int32, (BLQ, NA), 1)
            pick = jnp.sum(jnp.where(lane == seqs_ref[rows], aa, 0.0),
                           axis=1, keepdims=True)
            maskc = mask_ref[rows]
            blk_s = blk_s + jnp.sum((pick - lse) * maskc)
            blk_n = blk_n + jnp.sum(maskc)

        @pl.when(i == 0)
        def _():
            s_ref[bb, 0] = 0.0
            n_ref[bb, 0] = 0.0

        s_ref[bb, 0] += blk_s
        n_ref[bb, 0] += blk_n

    grid = (B, GI)
    etab_specs = [
        pl.BlockSpec((1, BLQ, K, NA * NA),
                     functools.partial(lambda q, b, i: (b, i * Q + q, 0, 0), q))
        for q in range(Q)
    ]
    out = pl.pallas_call(
        body,
        grid=grid,
        in_specs=etab_specs + [
            pl.BlockSpec((BL, KP), lambda b, i: (b * GI + i, 0)),
            pl.BlockSpec((BL, 1), lambda b, i: (b * GI + i, 0)),
            pl.BlockSpec((BL, 1), lambda b, i: (b * GI + i, 0)),
            pl.BlockSpec((NA * NA, NA), lambda b, i: (0, 0)),
        ],
        out_specs=[
            pl.BlockSpec((B, 1), lambda b, i: (0, 0),
                         memory_space=pltpu.SMEM),
            pl.BlockSpec((B, 1), lambda b, i: (0, 0),
                         memory_space=pltpu.SMEM),
        ],
        out_shape=[
            jax.ShapeDtypeStruct((B, 1), jnp.float32),
            jax.ShapeDtypeStruct((B, 1), jnp.float32),
        ],
    )(*([etab] * Q), eaa2d, seqs2d, mask2d, seg)
    return out


def kernel(etab, E_idx, seqs, x_mask):
    eidx_flat = jnp.pad(E_idx, ((0, 0), (0, 0), (0, KP - K))).reshape(-1)
    seqs_flat = seqs.reshape(-1)
    eaa = _sc_eaa(eidx_flat, seqs_flat)
    s, n = _tc_main(
        etab,
        eaa.reshape(B * L, KP),
        seqs_flat.reshape(B * L, 1),
        x_mask.reshape(B * L, 1).astype(jnp.float32),
        jnp.asarray(_SEG),
    )
    return -jnp.mean(s[:, 0] / n[:, 0])
